# baseline (speedup 1.0000x reference)
import functools

import jax
import jax.numpy as jnp
from jax import lax
from jax.experimental import pallas as pl
from jax.experimental.pallas import tpu as pltpu
from jax.experimental.pallas import tpu_sc as plsc

_NC, _NS = 2, 16
_NW = _NC * _NS
_C = 512


def _params(N, D, S):
    SPW = (-(-S // _NW) + 7) // 8 * 8
    SLICE = (SPW + 1 + 16 + 7) // 8 * 8
    PAD = (_NW - 1) * SPW + SLICE
    CB = _C + 8
    KMAX = -(-N // _C)
    KPAD = (KMAX + 1 + 16 + 7) // 8 * 8
    return SPW, SLICE, PAD, CB, KMAX, KPAD


def _build_seg_sum(N, D, S):
    SPW, SLICE, PAD, CB, KMAX, KPAD = _params(N, D, S)
    DV = D // 16

    mesh = plsc.VectorSubcoreMesh(
        core_axis_name="c", subcore_axis_name="s",
        num_cores=_NC, num_subcores=_NS)

    @functools.partial(
        pl.kernel,
        out_type=jax.ShapeDtypeStruct((_NW * SPW, D), jnp.float32),
        mesh=mesh,
        scratch_types=[
            pltpu.VMEM((SLICE,), jnp.int32),
            pltpu.VMEM((SLICE,), jnp.int32),
            pltpu.VMEM((KPAD,), jnp.int32),
            pltpu.VMEM((CB, D), jnp.float32),
            pltpu.VMEM((SPW, D), jnp.float32),
        ],
    )
    def seg_sum(x_hbm, starts_hbm, adv_hbm, pc_hbm, out_hbm,
                starts_v, adv_v, pc_v, rows_v, out_v):
        wid = lax.axis_index("s") * _NC + lax.axis_index("c")
        s_lo = wid * SPW
        n_s = jnp.maximum(0, jnp.minimum(SPW, S - s_lo))
        pltpu.sync_copy(starts_hbm.at[pl.ds(s_lo, SLICE)], starts_v)
        pltpu.sync_copy(adv_hbm.at[pl.ds(s_lo, SLICE)], adv_v)
        pltpu.sync_copy(pc_hbm.at[pl.ds(wid * KPAD, KPAD)], pc_v)

        def sread(ref, idx):
            return ref[pl.ds(idx, 16)][0]

        row_lo = sread(starts_v, 0)
        row_hi = sread(starts_v, n_s)
        n_chunks = (row_hi - row_lo + _C - 1) // _C
        zero = jnp.zeros((16,), jnp.float32)

        sptr_init = jnp.clip(sread(adv_v, 0) - s_lo, 0, SLICE - 17)
        nxt_init = sread(starts_v, sptr_init + 1)

        def zero_body(i, _):
            for j in range(DV):
                out_v[i, pl.ds(16 * j, 16)] = zero
            return 0

        lax.fori_loop(0, SPW, zero_body, 0)

        def chunk_base(k):
            kc = jnp.clip(k, 0, jnp.maximum(n_chunks - 1, 0))
            base = row_lo + kc * _C
            return pl.multiple_of(jnp.clip((base // 8) * 8, 0, N - CB), 8)

        def process_chunk(k, carry):
            r0, sptr0, nxt0, acc0 = carry[0], carry[1], carry[2], carry[3:]
            base = row_lo + k * _C
            shift = chunk_base(k)
            pltpu.sync_copy(x_hbm.at[pl.ds(shift, CB)], rows_v)
            chunk_end = jnp.minimum(base + _C, row_hi)
            pc = sread(pc_v, k)

            def piece_body(_, st):
                r, sptr, nxt = st[0], st[1], st[2]
                acc = st[3:]
                limit = jnp.minimum(nxt, chunk_end)

                def row_body(i, a):
                    return tuple(
                        a[j] + rows_v[i - shift, pl.ds(16 * j, 16)]
                        for j in range(DV))

                acc = plsc.parallel_loop(
                    r, limit, carry=acc, unroll=4)(row_body)
                do_flush = jnp.logical_and(limit == nxt, sptr < n_s)

                @pl.when(do_flush)
                def _():
                    for j in range(DV):
                        out_v[sptr, pl.ds(16 * j, 16)] = acc[j]

                jump = jnp.clip(sread(adv_v, sptr + 1) - s_lo, 0, SLICE - 17)
                sptr = jnp.where(do_flush, jump, sptr)
                nxt = sread(starts_v, sptr + 1)
                acc = tuple(jnp.where(do_flush, zero, a) for a in acc)
                return (limit, sptr, nxt) + acc

            return lax.fori_loop(0, pc, piece_body,
                                 (r0, sptr0, nxt0) + acc0)

        carry0 = (row_lo, sptr_init, nxt_init) + (zero,) * DV
        lax.fori_loop(0, n_chunks, process_chunk, carry0)

        pltpu.sync_copy(out_v, out_hbm.at[pl.ds(s_lo, SPW)])

    return seg_sum


def kernel(sentence_embeddings, sentence_spans):
    x = sentence_embeddings
    N, D = x.shape
    S = sentence_spans.shape[0]
    SPW, SLICE, PAD, CB, KMAX, KPAD = _params(N, D, S)

    starts = sentence_spans[:, 0].astype(jnp.int32)
    starts_ext = jnp.concatenate(
        [starts, jnp.full((PAD - S,), N, dtype=jnp.int32)])
    idx = jnp.arange(S, dtype=jnp.int32)
    last_of_run = jnp.concatenate(
        [starts[1:] != starts[:-1], jnp.ones((1,), bool)])
    last_occ = lax.cummin(jnp.where(last_of_run, idx, S),
                          axis=0, reverse=True)
    adv_ext = jnp.concatenate(
        [last_occ, jnp.full((PAD - S,), S, dtype=jnp.int32)])

    w = jnp.arange(_NW, dtype=jnp.int32)
    s_lo = w * SPW
    n_s = jnp.clip(S - s_lo, 0, SPW)
    row_lo = starts_ext[s_lo]
    row_hi = starts_ext[s_lo + n_s]
    n_chunks = -(-(row_hi - row_lo) // _C)
    s_idx = jnp.arange(S, dtype=jnp.int32)
    w_s = s_idx // SPW
    row_lo_s = jnp.repeat(row_lo, SPW)[:S]
    b_s = starts_ext[1:S + 1]
    k_s = (b_s - 1 - row_lo_s) // _C
    flat = jnp.where(b_s > row_lo_s, w_s * KPAD + k_s, _NW * KPAD)
    hist = jnp.zeros((_NW * KPAD + 1,), jnp.int32).at[flat].add(1)
    k = jnp.arange(KPAD, dtype=jnp.int32)
    active = (k[None, :] < n_chunks[:, None]).astype(jnp.int32)
    pc = (hist[:_NW * KPAD].reshape(_NW, KPAD) + active).reshape(-1)

    seg_sum = _build_seg_sum(N, D, S)
    out_pad = seg_sum(x, starts_ext, adv_ext, pc)
    return out_pad[:S]


# speedup vs baseline: 1.3888x; 1.3888x over previous
import functools

import jax
import jax.numpy as jnp
from jax import lax
from jax.experimental import pallas as pl
from jax.experimental.pallas import tpu as pltpu
from jax.experimental.pallas import tpu_sc as plsc

_NC, _NS = 2, 16
_NW = _NC * _NS
_C = 256


def _params(N, D, S):
    SPW = (-(-S // _NW) + 7) // 8 * 8
    SLICE = (SPW + 1 + 16 + 7) // 8 * 8
    PAD = (_NW - 1) * SPW + SLICE
    CB = _C + 8
    KMAX = -(-N // _C)
    KPAD = (KMAX + 1 + 16 + 7) // 8 * 8
    return SPW, SLICE, PAD, CB, KMAX, KPAD


def _build_seg_sum(N, D, S):
    SPW, SLICE, PAD, CB, KMAX, KPAD = _params(N, D, S)
    DV = D // 16

    mesh = plsc.VectorSubcoreMesh(
        core_axis_name="c", subcore_axis_name="s",
        num_cores=_NC, num_subcores=_NS)

    @functools.partial(
        pl.kernel,
        out_type=jax.ShapeDtypeStruct((_NW * SPW, D), jnp.float32),
        mesh=mesh,
        scratch_types=[
            pltpu.VMEM((SLICE,), jnp.int32),
            pltpu.VMEM((SLICE,), jnp.int32),
            pltpu.VMEM((KPAD,), jnp.int32),
            pltpu.VMEM((2 * CB, D), jnp.float32),
            pltpu.VMEM((SPW, D), jnp.float32),
            pltpu.SemaphoreType.DMA,
        ],
    )
    def seg_sum(x_hbm, starts_hbm, adv_hbm, pc_hbm, out_hbm,
                starts_v, adv_v, pc_v, rows_v, out_v, sem):
        wid = lax.axis_index("s") * _NC + lax.axis_index("c")
        s_lo = wid * SPW
        n_s = jnp.maximum(0, jnp.minimum(SPW, S - s_lo))
        pltpu.sync_copy(starts_hbm.at[pl.ds(s_lo, SLICE)], starts_v)
        pltpu.sync_copy(adv_hbm.at[pl.ds(s_lo, SLICE)], adv_v)
        pltpu.sync_copy(pc_hbm.at[pl.ds(wid * KPAD, KPAD)], pc_v)

        def sread(ref, idx):
            return ref[pl.ds(idx, 16)][0]

        row_lo = sread(starts_v, 0)
        row_hi = sread(starts_v, n_s)
        n_chunks = (row_hi - row_lo + _C - 1) // _C
        zero = jnp.zeros((16,), jnp.float32)

        sptr_init = jnp.clip(sread(adv_v, 0) - s_lo, 0, SLICE - 17)
        nxt_init = sread(starts_v, sptr_init + 1)

        def zero_body(i, _):
            for j in range(DV):
                out_v[i, pl.ds(16 * j, 16)] = zero
            return 0

        lax.fori_loop(0, SPW, zero_body, 0)

        def chunk_base(k):
            kc = jnp.clip(k, 0, jnp.maximum(n_chunks - 1, 0))
            base = row_lo + kc * _C
            return pl.multiple_of(jnp.clip((base // 8) * 8, 0, N - CB), 8)

        def buf_off(k):
            return pl.multiple_of((k % 2) * CB, 8)

        def start_copy(k):
            pltpu.async_copy(x_hbm.at[pl.ds(chunk_base(k), CB)],
                             rows_v.at[pl.ds(buf_off(k), CB)], sem)

        def wait_copy():
            pltpu.make_async_copy(x_hbm.at[pl.ds(0, CB)],
                                  rows_v.at[pl.ds(0, CB)], sem).wait()

        def process_chunk(k, carry):
            r0, sptr0, nxt0, acc0 = carry[0], carry[1], carry[2], carry[3:]
            base = row_lo + k * _C
            shift = chunk_base(k) - buf_off(k)
            chunk_end = jnp.minimum(base + _C, row_hi)
            pc = sread(pc_v, k)

            def piece_body(_, st):
                r, sptr, nxt = st[0], st[1], st[2]
                acc = st[3:]
                limit = jnp.minimum(nxt, chunk_end)

                def row_body(i, a):
                    return tuple(
                        a[j] + rows_v[i - shift, pl.ds(16 * j, 16)]
                        for j in range(DV))

                acc = plsc.parallel_loop(
                    r, limit, carry=acc, unroll=4)(row_body)
                do_flush = jnp.logical_and(limit == nxt, sptr < n_s)

                @pl.when(do_flush)
                def _():
                    for j in range(DV):
                        out_v[sptr, pl.ds(16 * j, 16)] = acc[j]

                jump = jnp.clip(sread(adv_v, sptr + 1) - s_lo, 0, SLICE - 17)
                sptr = jnp.where(do_flush, jump, sptr)
                nxt = sread(starts_v, sptr + 1)
                acc = tuple(jnp.where(do_flush, zero, a) for a in acc)
                return (limit, sptr, nxt) + acc

            return lax.fori_loop(0, pc, piece_body,
                                 (r0, sptr0, nxt0) + acc0)

        start_copy(0)

        def chunk_loop(k, carry):
            wait_copy()
            start_copy(k + 1)
            return process_chunk(k, carry)

        carry0 = (row_lo, sptr_init, nxt_init) + (zero,) * DV
        lax.fori_loop(0, n_chunks, chunk_loop, carry0)
        wait_copy()

        pltpu.sync_copy(out_v, out_hbm.at[pl.ds(s_lo, SPW)])

    return seg_sum


def kernel(sentence_embeddings, sentence_spans):
    x = sentence_embeddings
    N, D = x.shape
    S = sentence_spans.shape[0]
    SPW, SLICE, PAD, CB, KMAX, KPAD = _params(N, D, S)

    starts = sentence_spans[:, 0].astype(jnp.int32)
    starts_ext = jnp.concatenate(
        [starts, jnp.full((PAD - S,), N, dtype=jnp.int32)])
    idx = jnp.arange(S, dtype=jnp.int32)
    last_of_run = jnp.concatenate(
        [starts[1:] != starts[:-1], jnp.ones((1,), bool)])
    last_occ = lax.cummin(jnp.where(last_of_run, idx, S),
                          axis=0, reverse=True)
    adv_ext = jnp.concatenate(
        [last_occ, jnp.full((PAD - S,), S, dtype=jnp.int32)])

    w = jnp.arange(_NW, dtype=jnp.int32)
    s_lo = w * SPW
    n_s = jnp.clip(S - s_lo, 0, SPW)
    row_lo = starts_ext[s_lo]
    row_hi = starts_ext[s_lo + n_s]
    n_chunks = -(-(row_hi - row_lo) // _C)
    s_idx = jnp.arange(S, dtype=jnp.int32)
    w_s = s_idx // SPW
    row_lo_s = jnp.repeat(row_lo, SPW)[:S]
    b_s = starts_ext[1:S + 1]
    k_s = (b_s - 1 - row_lo_s) // _C
    flat = jnp.where(b_s > row_lo_s, w_s * KPAD + k_s, _NW * KPAD)
    hist = jnp.zeros((_NW * KPAD + 1,), jnp.int32).at[flat].add(1)
    k = jnp.arange(KPAD, dtype=jnp.int32)
    active = (k[None, :] < n_chunks[:, None]).astype(jnp.int32)
    pc = (hist[:_NW * KPAD].reshape(_NW, KPAD) + active).reshape(-1)

    seg_sum = _build_seg_sum(N, D, S)
    out_pad = seg_sum(x, starts_ext, adv_ext, pc)
    return out_pad[:S]
